# Initial kernel scaffold; baseline (speedup 1.0000x reference)
import jax
import jax.numpy as jnp
from jax import lax
from jax.experimental import pallas as pl
from jax.experimental.pallas import tpu as pltpu
from jax.experimental.pallas import tpu_sc as plsc

_NODE_NUM = 100000
_REL_NUM = 1000
_D = 128
_K = 20
_P = 16384
_NS = 2 * _P
_NE = 2 * _NS
_MARGIN = 1.0

_NW = 32
_EPW = _NE // _NW
_SG = 32
_NSG = _EPW // _SG
_NIDXROWS = _NE * _K // 128
_IPW = _NIDXROWS // _NW
_SCHUNKS = _EPW // 128
_RPW = _NS // _NW
_RCHUNKS = _RPW // 128


def _sc_body(selfidx_hbm, nghidx_hbm, relidx_hbm, node_hbm, rel_hbm,
             out_self, out_agg, out_rel,
             nghidx_v, selfidx_v, relidx_v, nbuf, agg_v, rowbuf, sem):
    c = lax.axis_index("c")
    s = lax.axis_index("s")
    w = s * 2 + c
    ebase = w * _EPW

    def supergroup(g, carry):
        pltpu.sync_copy(nghidx_hbm.at[pl.ds(w * _IPW + g * 5, 5)], nghidx_v)
        handles = []
        for j in range(5):
            handles.append(pltpu.async_copy(
                node_hbm.at[nghidx_v.at[j]],
                nbuf.at[pl.ds(j * 128, 128)], sem))
        for h in handles:
            h.wait()

        def sample_body(i, carry2):
            base = i * _K
            acc = tuple(nbuf[base, pl.ds(j * 16, 16)] for j in range(8))

            def kbody(k, accs):
                return tuple(accs[j] + nbuf[base + k, pl.ds(j * 16, 16)]
                             for j in range(8))

            acc = lax.fori_loop(1, _K, kbody, acc)
            for j in range(8):
                agg_v[i, pl.ds(j * 16, 16)] = acc[j]
            return carry2

        lax.fori_loop(0, _SG, sample_body, 0)
        pltpu.sync_copy(agg_v, out_agg.at[pl.ds(ebase + g * _SG, _SG)])
        return carry

    lax.fori_loop(0, _NSG, supergroup, 0)

    def self_chunk(t, carry):
        pltpu.sync_copy(selfidx_hbm.at[w * _SCHUNKS + t], selfidx_v)
        pltpu.async_copy(node_hbm.at[selfidx_v], rowbuf, sem).wait()
        pltpu.sync_copy(rowbuf, out_self.at[pl.ds(ebase + t * 128, 128)])
        return carry

    lax.fori_loop(0, _SCHUNKS, self_chunk, 0)

    def rel_chunk(t, carry):
        pltpu.sync_copy(relidx_hbm.at[w * _RCHUNKS + t], relidx_v)
        pltpu.async_copy(rel_hbm.at[relidx_v], rowbuf, sem).wait()
        pltpu.sync_copy(rowbuf, out_rel.at[pl.ds(w * _RPW + t * 128, 128)])
        return carry

    lax.fori_loop(0, _RCHUNKS, rel_chunk, 0)


def _sc_gather(selfidx, nghidx, relidx, node_emb, rel_emb):
    mesh = plsc.VectorSubcoreMesh(core_axis_name="c", subcore_axis_name="s")
    return pl.kernel(
        _sc_body,
        out_type=(
            jax.ShapeDtypeStruct((_NE, _D), jnp.float32),
            jax.ShapeDtypeStruct((_NE, _D), jnp.float32),
            jax.ShapeDtypeStruct((_NS, _D), jnp.float32),
        ),
        mesh=mesh,
        scratch_types=[
            pltpu.VMEM((5, 128), jnp.int32),
            pltpu.VMEM((128,), jnp.int32),
            pltpu.VMEM((128,), jnp.int32),
            pltpu.VMEM((_SG * _K, _D), jnp.float32),
            pltpu.VMEM((_SG, _D), jnp.float32),
            pltpu.VMEM((128, _D), jnp.float32),
            pltpu.SemaphoreType.DMA,
        ],
    )(selfidx, nghidx, relidx, node_emb, rel_emb)


_BP = 512
_NBLK = _P // _BP


def _tc_body(self_sp, self_sn, self_dp, self_dn,
             agg_sp, agg_sn, agg_dp, agg_dn,
             rel_p, rel_n, w1, w2, bvec, out):
    i = pl.program_id(0)

    def enc(se, ag):
        h = lax.dot_general(se[...], w1[...], (((1,), (0,)), ((), ())),
                            precision=lax.Precision.HIGHEST,
                            preferred_element_type=jnp.float32)
        h = h + lax.dot_general(ag[...], w2[...], (((1,), (0,)), ((), ())),
                                precision=lax.Precision.HIGHEST,
                                preferred_element_type=jnp.float32)
        return jnp.maximum(h + bvec[...], 0.0)

    hsp = enc(self_sp, agg_sp)
    hsn = enc(self_sn, agg_sn)
    hdp = enc(self_dp, agg_dp)
    hdn = enc(self_dn, agg_dn)
    dp = hsp + rel_p[...] - hdp
    dn = hsn + rel_n[...] - hdn
    sp = -jnp.sqrt(jnp.sum(dp * dp, axis=1) + 1e-12)
    sn = -jnp.sqrt(jnp.sum(dn * dn, axis=1) + 1e-12)
    part = jnp.sum(jnp.maximum(0.0, sn - sp + _MARGIN)) * (1.0 / _P)

    @pl.when(i == 0)
    def _():
        out[0, 0] = 0.0

    out[0, 0] += part


def _tc_dense(out_self, out_agg, out_rel, w1, w2, bvec):
    row_spec = lambda off: pl.BlockSpec((_BP, _D), lambda i, o=off: (i + o, 0))
    loss = pl.pallas_call(
        _tc_body,
        grid=(_NBLK,),
        in_specs=[
            row_spec(0), row_spec(_NBLK), row_spec(2 * _NBLK), row_spec(3 * _NBLK),
            row_spec(0), row_spec(_NBLK), row_spec(2 * _NBLK), row_spec(3 * _NBLK),
            row_spec(0), row_spec(_NBLK),
            pl.BlockSpec((_D, _D), lambda i: (0, 0)),
            pl.BlockSpec((_D, _D), lambda i: (0, 0)),
            pl.BlockSpec((1, _D), lambda i: (0, 0)),
        ],
        out_specs=pl.BlockSpec((1, 1), lambda i: (0, 0),
                               memory_space=pltpu.SMEM),
        out_shape=jax.ShapeDtypeStruct((1, 1), jnp.float32),
        compiler_params=pltpu.CompilerParams(
            dimension_semantics=("arbitrary",)),
    )(out_self, out_self, out_self, out_self,
      out_agg, out_agg, out_agg, out_agg,
      out_rel, out_rel, w1, w2, bvec)
    return loss[0, 0]


def kernel(train_pos, train_neg, ngh_idx_src, ngh_idx_dst,
           node_emb, rel_emb, W, b):
    alls = jnp.concatenate([train_pos, train_neg], axis=0).astype(jnp.int32)
    src = alls[:, 0] % _NODE_NUM
    dst = alls[:, 1] % _NODE_NUM
    rel = alls[:, 2] % _REL_NUM
    selfidx = jnp.concatenate([src, dst]).reshape(_NE // 128, 128)
    nghidx = jnp.concatenate(
        [ngh_idx_src, ngh_idx_dst], axis=0).astype(jnp.int32).reshape(
            _NIDXROWS, 128)
    relidx = rel.reshape(_NS // 128, 128)

    out_self, out_agg, out_rel = _sc_gather(
        selfidx, nghidx, relidx, node_emb, rel_emb)

    w1 = W[:_D]
    w2 = W[_D:] * (1.0 / _K)
    bvec = b.reshape(1, _D)
    return _tc_dense(out_self, out_agg, out_rel, w1, w2, bvec)


# speedup vs baseline: 9.3941x; 9.3941x over previous
import jax
import jax.numpy as jnp
from jax import lax
from jax.experimental import pallas as pl
from jax.experimental.pallas import tpu as pltpu
from jax.experimental.pallas import tpu_sc as plsc

_NODE_NUM = 100000
_REL_NUM = 1000
_D = 128
_K = 20
_P = 16384
_NS = 2 * _P
_NE = 2 * _NS
_MARGIN = 1.0

_NW = 32
_EPW = _NE // _NW
_SG = 32
_NSG = _EPW // _SG
_NIDXROWS = _NE * _K // 128
_IPW = _NIDXROWS // _NW
_SCHUNKS = _EPW // 128
_RPW = _NS // _NW
_RCHUNKS = _RPW // 128


def _sc_body(selfidx_hbm, nghidx_hbm, relidx_hbm, node_hbm, rel_hbm,
             out_self, out_agg, out_rel,
             nghidx_v, selfidx_v, relidx_v, nbuf, agg_v, rowbuf, sem):
    c = lax.axis_index("c")
    s = lax.axis_index("s")
    w = s * 2 + c
    ebase = w * _EPW

    def block(bb, carry0):
        pltpu.sync_copy(nghidx_hbm.at[pl.ds(w * _IPW + bb * 40, 40)],
                        nghidx_v)

        def supergroup(sg, carry):
            handles = []
            for j in range(5):
                handles.append(pltpu.async_copy(
                    node_hbm.at[nghidx_v.at[sg * 5 + j]],
                    nbuf.at[pl.ds(j * 128, 128)], sem))
            for h in handles:
                h.wait()

            def sample_body(i, carry2):
                base = i * _K
                acc = tuple(nbuf[base, pl.ds(j * 16, 16)] for j in range(8))

                def kbody(k, accs):
                    return tuple(accs[j] + nbuf[base + k, pl.ds(j * 16, 16)]
                                 for j in range(8))

                acc = lax.fori_loop(1, _K, kbody, acc)
                for j in range(8):
                    agg_v[i, pl.ds(j * 16, 16)] = acc[j]
                return carry2

            lax.fori_loop(0, _SG, sample_body, 0)
            pltpu.sync_copy(
                agg_v,
                out_agg.at[pl.ds(ebase + bb * (8 * _SG) + sg * _SG, _SG)])
            return carry

        lax.fori_loop(0, 8, supergroup, 0)
        return carry0

    lax.fori_loop(0, _NSG // 8, block, 0)

    pltpu.sync_copy(selfidx_hbm.at[pl.ds(w * _SCHUNKS, _SCHUNKS)], selfidx_v)

    def self_chunk(t, carry):
        pltpu.async_copy(node_hbm.at[selfidx_v.at[t]], rowbuf, sem).wait()
        pltpu.sync_copy(rowbuf, out_self.at[pl.ds(ebase + t * 128, 128)])
        return carry

    lax.fori_loop(0, _SCHUNKS, self_chunk, 0)

    pltpu.sync_copy(relidx_hbm.at[pl.ds(w * _RCHUNKS, _RCHUNKS)], relidx_v)

    def rel_chunk(t, carry):
        pltpu.async_copy(rel_hbm.at[relidx_v.at[t]], rowbuf, sem).wait()
        pltpu.sync_copy(rowbuf, out_rel.at[pl.ds(w * _RPW + t * 128, 128)])
        return carry

    lax.fori_loop(0, _RCHUNKS, rel_chunk, 0)


def _sc_gather(selfidx, nghidx, relidx, node_emb, rel_emb):
    mesh = plsc.VectorSubcoreMesh(core_axis_name="c", subcore_axis_name="s")
    return pl.kernel(
        _sc_body,
        out_type=(
            jax.ShapeDtypeStruct((_NE, _D), jnp.float32),
            jax.ShapeDtypeStruct((_NE, _D), jnp.float32),
            jax.ShapeDtypeStruct((_NS, _D), jnp.float32),
        ),
        mesh=mesh,
        scratch_types=[
            pltpu.VMEM((40, 128), jnp.int32),
            pltpu.VMEM((_SCHUNKS, 128), jnp.int32),
            pltpu.VMEM((_RCHUNKS, 128), jnp.int32),
            pltpu.VMEM((_SG * _K, _D), jnp.float32),
            pltpu.VMEM((_SG, _D), jnp.float32),
            pltpu.VMEM((128, _D), jnp.float32),
            pltpu.SemaphoreType.DMA,
        ],
    )(selfidx, nghidx, relidx, node_emb, rel_emb)


_BP = 512
_NBLK = _P // _BP


def _tc_body(self_sp, self_sn, self_dp, self_dn,
             agg_sp, agg_sn, agg_dp, agg_dn,
             rel_p, rel_n, w1, w2, bvec, out):
    i = pl.program_id(0)

    def enc(se, ag):
        h = lax.dot_general(se[...], w1[...], (((1,), (0,)), ((), ())),
                            precision=lax.Precision.HIGHEST,
                            preferred_element_type=jnp.float32)
        h = h + lax.dot_general(ag[...], w2[...], (((1,), (0,)), ((), ())),
                                precision=lax.Precision.HIGHEST,
                                preferred_element_type=jnp.float32)
        return jnp.maximum(h + bvec[...], 0.0)

    hsp = enc(self_sp, agg_sp)
    hsn = enc(self_sn, agg_sn)
    hdp = enc(self_dp, agg_dp)
    hdn = enc(self_dn, agg_dn)
    dp = hsp + rel_p[...] - hdp
    dn = hsn + rel_n[...] - hdn
    sp = -jnp.sqrt(jnp.sum(dp * dp, axis=1) + 1e-12)
    sn = -jnp.sqrt(jnp.sum(dn * dn, axis=1) + 1e-12)
    part = jnp.sum(jnp.maximum(0.0, sn - sp + _MARGIN)) * (1.0 / _P)

    @pl.when(i == 0)
    def _():
        out[0, 0] = 0.0

    out[0, 0] += part


def _tc_dense(out_self, out_agg, out_rel, w1, w2, bvec):
    row_spec = lambda off: pl.BlockSpec((_BP, _D), lambda i, o=off: (i + o, 0))
    loss = pl.pallas_call(
        _tc_body,
        grid=(_NBLK,),
        in_specs=[
            row_spec(0), row_spec(_NBLK), row_spec(2 * _NBLK), row_spec(3 * _NBLK),
            row_spec(0), row_spec(_NBLK), row_spec(2 * _NBLK), row_spec(3 * _NBLK),
            row_spec(0), row_spec(_NBLK),
            pl.BlockSpec((_D, _D), lambda i: (0, 0)),
            pl.BlockSpec((_D, _D), lambda i: (0, 0)),
            pl.BlockSpec((1, _D), lambda i: (0, 0)),
        ],
        out_specs=pl.BlockSpec((1, 1), lambda i: (0, 0),
                               memory_space=pltpu.SMEM),
        out_shape=jax.ShapeDtypeStruct((1, 1), jnp.float32),
        compiler_params=pltpu.CompilerParams(
            dimension_semantics=("arbitrary",)),
    )(out_self, out_self, out_self, out_self,
      out_agg, out_agg, out_agg, out_agg,
      out_rel, out_rel, w1, w2, bvec)
    return loss[0, 0]


def kernel(train_pos, train_neg, ngh_idx_src, ngh_idx_dst,
           node_emb, rel_emb, W, b):
    alls = jnp.concatenate([train_pos, train_neg], axis=0).astype(jnp.int32)
    src = alls[:, 0] % _NODE_NUM
    dst = alls[:, 1] % _NODE_NUM
    rel = alls[:, 2] % _REL_NUM
    selfidx = jnp.concatenate([src, dst]).reshape(_NE // 128, 128)
    nghidx = jnp.concatenate(
        [ngh_idx_src, ngh_idx_dst], axis=0).astype(jnp.int32).reshape(
            _NIDXROWS, 128)
    relidx = rel.reshape(_NS // 128, 128)

    out_self, out_agg, out_rel = _sc_gather(
        selfidx, nghidx, relidx, node_emb, rel_emb)

    w1 = W[:_D]
    w2 = W[_D:] * (1.0 / _K)
    bvec = b.reshape(1, _D)
    return _tc_dense(out_self, out_agg, out_rel, w1, w2, bvec)


# speedup vs baseline: 13.3921x; 1.4256x over previous
import jax
import jax.numpy as jnp
from jax import lax
from jax.experimental import pallas as pl
from jax.experimental.pallas import tpu as pltpu
from jax.experimental.pallas import tpu_sc as plsc

_NODE_NUM = 100000
_REL_NUM = 1000
_D = 128
_K = 20
_P = 16384
_NS = 2 * _P
_NE = 2 * _NS
_MARGIN = 1.0

_NW = 32
_HG = 16
_NHG = _NE // _HG
_HGPW = _NHG // _NW
_NPAIR = _HGPW // 2
_BATCH = 32
_NBATCH = _HGPW // _BATCH
_NROWS = _HG * _K
_RPW = _NS // _NW
_RCHUNKS = _RPW // 128


def _sc_body(idx5_hbm, relidx_hbm, node_hbm, rel_hbm,
             out_enc, out_rel,
             idx0, idx1, nbufA, nbufB, obufA, obufB, relidx_v,
             semA, semB, semI, semOutA, semOutB):
    c = lax.axis_index("c")
    s = lax.axis_index("s")
    w = s * 2 + c
    hbase = w * _HGPW

    def issue_gathers(ibuf, gl, nbuf, sem):
        pltpu.async_copy(node_hbm.at[ibuf.at[3 * gl]],
                         nbuf.at[pl.ds(0, 128)], sem)
        pltpu.async_copy(node_hbm.at[ibuf.at[3 * gl + 1]],
                         nbuf.at[pl.ds(128, 128)], sem)
        pltpu.async_copy(node_hbm.at[ibuf.at[3 * gl + 2, pl.ds(0, 64)]],
                         nbuf.at[pl.ds(256, 64)], sem)
        pltpu.async_copy(node_hbm.at[ibuf.at[3 * gl + 2, pl.ds(64, _HG)]],
                         nbuf.at[pl.ds(_NROWS, _HG)], sem)

    def issue_by_parity(par, gl, nbuf, sem):
        @pl.when(par == 0)
        def _():
            issue_gathers(idx0, gl, nbuf, sem)

        @pl.when(par == 1)
        def _():
            issue_gathers(idx1, gl, nbuf, sem)

    def wait_gathers(nbuf, sem):
        pltpu.make_async_copy(node_hbm.at[idx0.at[0]],
                              nbuf.at[pl.ds(0, 128)], sem).wait()
        pltpu.make_async_copy(node_hbm.at[idx0.at[1]],
                              nbuf.at[pl.ds(128, 128)], sem).wait()
        pltpu.make_async_copy(node_hbm.at[idx0.at[2, pl.ds(0, 64)]],
                              nbuf.at[pl.ds(256, 64)], sem).wait()
        pltpu.make_async_copy(node_hbm.at[idx0.at[2, pl.ds(64, _HG)]],
                              nbuf.at[pl.ds(_NROWS, _HG)], sem).wait()

    def sum_half(nbuf, obuf):
        def sample_body(i, carry2):
            base = i * _K
            acc = tuple(nbuf[base, pl.ds(j * 16, 16)] for j in range(8))

            def kbody(k, accs):
                return tuple(accs[j] + nbuf[base + k, pl.ds(j * 16, 16)]
                             for j in range(8))

            acc = lax.fori_loop(1, _K, kbody, acc)
            for j in range(8):
                obuf[i, pl.ds(_D + j * 16, 16)] = acc[j]
                obuf[i, pl.ds(j * 16, 16)] = nbuf[_NROWS + i, pl.ds(j * 16, 16)]
            return carry2

        lax.fori_loop(0, _HG, sample_body, 0)

    def wait_out(obuf, sem):
        pltpu.make_async_copy(obuf, out_enc.at[pl.ds(0, _HG)], sem).wait()

    def stage_idx(bi, ibuf, sem):
        return pltpu.async_copy(
            idx5_hbm.at[pl.ds(3 * hbase + bi * (3 * _BATCH), 3 * _BATCH)],
            ibuf, sem)

    stage_idx(0, idx0, semI).wait()
    stage_idx(1, idx1, semI)
    issue_gathers(idx0, 0, nbufA, semA)

    def pair_body(t, carry):
        par = lax.rem(lax.div(t, 16), 2)
        gl = lax.rem(2 * t, _BATCH)
        hgA = hbase + 2 * t

        issue_by_parity(par, gl + 1, nbufB, semB)

        wait_gathers(nbufA, semA)

        @pl.when(t > 0)
        def _():
            wait_out(obufA, semOutA)

        sum_half(nbufA, obufA)
        pltpu.async_copy(obufA, out_enc.at[pl.ds(hgA * _HG, _HG)], semOutA)

        wait_gathers(nbufB, semB)

        nxt = 2 * t + 2
        boundary = lax.rem(nxt, _BATCH) == 0

        @pl.when(jnp.logical_and(boundary, nxt < _HGPW))
        def _():
            pltpu.make_async_copy(idx5_hbm.at[pl.ds(3 * hbase, 3 * _BATCH)],
                                  idx0, semI).wait()

        bi_next2 = lax.div(t, 16) + 2

        @pl.when(jnp.logical_and(boundary, bi_next2 <= _NBATCH - 1))
        def _():
            @pl.when(par == 0)
            def _():
                stage_idx(bi_next2, idx0, semI)

            @pl.when(par == 1)
            def _():
                stage_idx(bi_next2, idx1, semI)

        par_next = lax.rem(lax.div(t + 1, 16), 2)

        @pl.when(nxt < _HGPW)
        def _():
            issue_by_parity(par_next, lax.rem(nxt, _BATCH), nbufA, semA)

        @pl.when(t > 0)
        def _():
            wait_out(obufB, semOutB)

        sum_half(nbufB, obufB)
        pltpu.async_copy(obufB, out_enc.at[pl.ds((hgA + 1) * _HG, _HG)],
                         semOutB)
        return carry

    lax.fori_loop(0, _NPAIR, pair_body, 0)
    wait_out(obufA, semOutA)
    wait_out(obufB, semOutB)

    pltpu.async_copy(relidx_hbm.at[pl.ds(w * _RCHUNKS, _RCHUNKS)],
                     relidx_v, semI).wait()
    pltpu.async_copy(rel_hbm.at[relidx_v.at[0]],
                     nbufA.at[pl.ds(0, 128)], semA)

    def rel_chunk(t, carry):
        @pl.when(t + 1 < _RCHUNKS)
        def _():
            @pl.when(lax.rem(t, 2) == 0)
            def _():
                pltpu.async_copy(rel_hbm.at[relidx_v.at[t + 1]],
                                 nbufB.at[pl.ds(0, 128)], semB)

            @pl.when(lax.rem(t, 2) == 1)
            def _():
                pltpu.async_copy(rel_hbm.at[relidx_v.at[t + 1]],
                                 nbufA.at[pl.ds(0, 128)], semA)

        @pl.when(lax.rem(t, 2) == 0)
        def _():
            pltpu.make_async_copy(rel_hbm.at[relidx_v.at[0]],
                                  nbufA.at[pl.ds(0, 128)], semA).wait()
            pltpu.sync_copy(nbufA.at[pl.ds(0, 128)],
                            out_rel.at[pl.ds(w * _RPW + t * 128, 128)])

        @pl.when(lax.rem(t, 2) == 1)
        def _():
            pltpu.make_async_copy(rel_hbm.at[relidx_v.at[0]],
                                  nbufB.at[pl.ds(0, 128)], semB).wait()
            pltpu.sync_copy(nbufB.at[pl.ds(0, 128)],
                            out_rel.at[pl.ds(w * _RPW + t * 128, 128)])
        return carry

    lax.fori_loop(0, _RCHUNKS, rel_chunk, 0)


def _sc_gather(idx5, relidx, node_emb, rel_emb):
    mesh = plsc.VectorSubcoreMesh(core_axis_name="c", subcore_axis_name="s")
    return pl.kernel(
        _sc_body,
        out_type=(
            jax.ShapeDtypeStruct((_NE, 2 * _D), jnp.float32),
            jax.ShapeDtypeStruct((_NS, _D), jnp.float32),
        ),
        mesh=mesh,
        scratch_types=[
            pltpu.VMEM((3 * _BATCH, 128), jnp.int32),
            pltpu.VMEM((3 * _BATCH, 128), jnp.int32),
            pltpu.VMEM((_NROWS + _HG, _D), jnp.float32),
            pltpu.VMEM((_NROWS + _HG, _D), jnp.float32),
            pltpu.VMEM((_HG, 2 * _D), jnp.float32),
            pltpu.VMEM((_HG, 2 * _D), jnp.float32),
            pltpu.VMEM((_RCHUNKS, 128), jnp.int32),
            pltpu.SemaphoreType.DMA,
            pltpu.SemaphoreType.DMA,
            pltpu.SemaphoreType.DMA,
            pltpu.SemaphoreType.DMA,
            pltpu.SemaphoreType.DMA,
        ],
    )(idx5, relidx, node_emb, rel_emb)


_BP = 512
_NBLK = _P // _BP


def _tc_body(enc_sp, enc_sn, enc_dp, enc_dn,
             rel_p, rel_n, wf, bvec, out):
    i = pl.program_id(0)

    def enc(x):
        h = lax.dot_general(x[...], wf[...], (((1,), (0,)), ((), ())),
                            precision=lax.Precision.HIGHEST,
                            preferred_element_type=jnp.float32)
        return jnp.maximum(h + bvec[...], 0.0)

    hsp = enc(enc_sp)
    hsn = enc(enc_sn)
    hdp = enc(enc_dp)
    hdn = enc(enc_dn)
    dp = hsp + rel_p[...] - hdp
    dn = hsn + rel_n[...] - hdn
    sp = -jnp.sqrt(jnp.sum(dp * dp, axis=1) + 1e-12)
    sn = -jnp.sqrt(jnp.sum(dn * dn, axis=1) + 1e-12)
    part = jnp.sum(jnp.maximum(0.0, sn - sp + _MARGIN)) * (1.0 / _P)

    @pl.when(i == 0)
    def _():
        out[0, 0] = 0.0

    out[0, 0] += part


def _tc_dense(out_enc, out_rel, wf, bvec):
    enc_spec = lambda off: pl.BlockSpec((_BP, 2 * _D),
                                        lambda i, o=off: (i + o, 0))
    rel_spec = lambda off: pl.BlockSpec((_BP, _D),
                                        lambda i, o=off: (i + o, 0))
    loss = pl.pallas_call(
        _tc_body,
        grid=(_NBLK,),
        in_specs=[
            enc_spec(0), enc_spec(_NBLK), enc_spec(2 * _NBLK),
            enc_spec(3 * _NBLK),
            rel_spec(0), rel_spec(_NBLK),
            pl.BlockSpec((2 * _D, _D), lambda i: (0, 0)),
            pl.BlockSpec((1, _D), lambda i: (0, 0)),
        ],
        out_specs=pl.BlockSpec((1, 1), lambda i: (0, 0),
                               memory_space=pltpu.SMEM),
        out_shape=jax.ShapeDtypeStruct((1, 1), jnp.float32),
        compiler_params=pltpu.CompilerParams(
            dimension_semantics=("arbitrary",)),
    )(out_enc, out_enc, out_enc, out_enc,
      out_rel, out_rel, wf, bvec)
    return loss[0, 0]


def kernel(train_pos, train_neg, ngh_idx_src, ngh_idx_dst,
           node_emb, rel_emb, W, b):
    alls = jnp.concatenate([train_pos, train_neg], axis=0).astype(jnp.int32)
    src = alls[:, 0] % _NODE_NUM
    dst = alls[:, 1] % _NODE_NUM
    rel = alls[:, 2] % _REL_NUM

    selfidx = jnp.concatenate([src, dst]).reshape(_NHG, _HG)
    nghidx = jnp.concatenate(
        [ngh_idx_src, ngh_idx_dst], axis=0).astype(jnp.int32).reshape(
            _NHG, _NROWS)
    pad = jnp.zeros((_NHG, 384 - _NROWS - _HG), dtype=jnp.int32)
    idx5 = jnp.concatenate([nghidx, selfidx, pad], axis=1).reshape(
        3 * _NHG, 128)
    relidx = rel.reshape(_NS // 128, 128)

    out_enc, out_rel = _sc_gather(idx5, relidx, node_emb, rel_emb)

    wf = jnp.concatenate([W[:_D], W[_D:] * (1.0 / _K)], axis=0)
    bvec = b.reshape(1, _D)
    return _tc_dense(out_enc, out_rel, wf, bvec)


# speedup vs baseline: 14.7847x; 1.1040x over previous
import jax
import jax.numpy as jnp
from jax import lax
from jax.experimental import pallas as pl
from jax.experimental.pallas import tpu as pltpu
from jax.experimental.pallas import tpu_sc as plsc

_NODE_NUM = 100000
_REL_NUM = 1000
_D = 128
_K = 20
_P = 16384
_NS = 2 * _P
_NE = 2 * _NS
_MARGIN = 1.0

_NW = 32
_HG = 16
_NHG = _NE // _HG
_HGPW = _NHG // _NW
_NPAIR = _HGPW // 2
_BATCH = 32
_NBATCH = _HGPW // _BATCH
_NROWS = _HG * _K
_RPW = _NS // _NW
_RCHUNKS = _RPW // 128


def _sc_body(idx5_hbm, relidx_hbm, node_hbm, rel_hbm,
             out_enc, out_rel,
             idx0, idx1, nbufA, nbufB, obufA, obufB, relidx_v,
             semA, semB, semI, semOutA, semOutB):
    c = lax.axis_index("c")
    s = lax.axis_index("s")
    w = s * 2 + c
    hbase = w * _HGPW

    def issue_gathers(ibuf, gl, nbuf, sem):
        pltpu.async_copy(node_hbm.at[ibuf.at[3 * gl]],
                         nbuf.at[pl.ds(0, 128)], sem)
        pltpu.async_copy(node_hbm.at[ibuf.at[3 * gl + 1]],
                         nbuf.at[pl.ds(128, 128)], sem)
        pltpu.async_copy(node_hbm.at[ibuf.at[3 * gl + 2, pl.ds(0, 64)]],
                         nbuf.at[pl.ds(256, 64)], sem)
        pltpu.async_copy(node_hbm.at[ibuf.at[3 * gl + 2, pl.ds(64, _HG)]],
                         nbuf.at[pl.ds(_NROWS, _HG)], sem)

    def issue_by_parity(par, gl, nbuf, sem):
        @pl.when(par == 0)
        def _():
            issue_gathers(idx0, gl, nbuf, sem)

        @pl.when(par == 1)
        def _():
            issue_gathers(idx1, gl, nbuf, sem)

    def wait_gathers(nbuf, sem):
        pltpu.make_async_copy(node_hbm.at[idx0.at[0]],
                              nbuf.at[pl.ds(0, 128)], sem).wait()
        pltpu.make_async_copy(node_hbm.at[idx0.at[1]],
                              nbuf.at[pl.ds(128, 128)], sem).wait()
        pltpu.make_async_copy(node_hbm.at[idx0.at[2, pl.ds(0, 64)]],
                              nbuf.at[pl.ds(256, 64)], sem).wait()
        pltpu.make_async_copy(node_hbm.at[idx0.at[2, pl.ds(64, _HG)]],
                              nbuf.at[pl.ds(_NROWS, _HG)], sem).wait()

    def sum_half(nbuf, obuf):
        def sample_body(i, carry2):
            base = i * _K
            acc = tuple(nbuf[base, pl.ds(j * 16, 16)] for j in range(8))

            def kbody(k, accs):
                return tuple(accs[j] + nbuf[base + k, pl.ds(j * 16, 16)]
                             for j in range(8))

            acc = lax.fori_loop(1, _K, kbody, acc)
            for j in range(8):
                obuf[i, pl.ds(_D + j * 16, 16)] = acc[j]
                obuf[i, pl.ds(j * 16, 16)] = nbuf[_NROWS + i, pl.ds(j * 16, 16)]
            return carry2

        lax.fori_loop(0, _HG, sample_body, 0)

    def wait_out(obuf, sem):
        pltpu.make_async_copy(obuf, out_enc.at[pl.ds(0, _HG)], sem).wait()

    def stage_idx(bi, ibuf, sem):
        return pltpu.async_copy(
            idx5_hbm.at[pl.ds(3 * hbase + bi * (3 * _BATCH), 3 * _BATCH)],
            ibuf, sem)

    stage_idx(0, idx0, semI).wait()
    stage_idx(1, idx1, semI)
    issue_gathers(idx0, 0, nbufA, semA)

    def pair_body(t, carry):
        par = lax.rem(lax.div(t, 16), 2)
        gl = lax.rem(2 * t, _BATCH)
        hgA = hbase + 2 * t

        issue_by_parity(par, gl + 1, nbufB, semB)

        wait_gathers(nbufA, semA)

        @pl.when(t > 0)
        def _():
            wait_out(obufA, semOutA)

        sum_half(nbufA, obufA)
        pltpu.async_copy(obufA, out_enc.at[pl.ds(hgA * _HG, _HG)], semOutA)

        wait_gathers(nbufB, semB)

        nxt = 2 * t + 2
        boundary = lax.rem(nxt, _BATCH) == 0

        @pl.when(jnp.logical_and(boundary, nxt < _HGPW))
        def _():
            pltpu.make_async_copy(idx5_hbm.at[pl.ds(3 * hbase, 3 * _BATCH)],
                                  idx0, semI).wait()

        bi_next2 = lax.div(t, 16) + 2

        @pl.when(jnp.logical_and(boundary, bi_next2 <= _NBATCH - 1))
        def _():
            @pl.when(par == 0)
            def _():
                stage_idx(bi_next2, idx0, semI)

            @pl.when(par == 1)
            def _():
                stage_idx(bi_next2, idx1, semI)

        par_next = lax.rem(lax.div(t + 1, 16), 2)

        @pl.when(nxt < _HGPW)
        def _():
            issue_by_parity(par_next, lax.rem(nxt, _BATCH), nbufA, semA)

        @pl.when(t > 0)
        def _():
            wait_out(obufB, semOutB)

        sum_half(nbufB, obufB)
        pltpu.async_copy(obufB, out_enc.at[pl.ds((hgA + 1) * _HG, _HG)],
                         semOutB)
        return carry

    lax.fori_loop(0, _NPAIR, pair_body, 0)
    wait_out(obufA, semOutA)
    wait_out(obufB, semOutB)

    pltpu.async_copy(relidx_hbm.at[pl.ds(w * _RCHUNKS, _RCHUNKS)],
                     relidx_v, semI).wait()
    pltpu.async_copy(rel_hbm.at[relidx_v.at[0]],
                     nbufA.at[pl.ds(0, 128)], semA)

    def rel_chunk(t, carry):
        @pl.when(t + 1 < _RCHUNKS)
        def _():
            @pl.when(lax.rem(t, 2) == 0)
            def _():
                pltpu.async_copy(rel_hbm.at[relidx_v.at[t + 1]],
                                 nbufB.at[pl.ds(0, 128)], semB)

            @pl.when(lax.rem(t, 2) == 1)
            def _():
                pltpu.async_copy(rel_hbm.at[relidx_v.at[t + 1]],
                                 nbufA.at[pl.ds(0, 128)], semA)

        @pl.when(lax.rem(t, 2) == 0)
        def _():
            pltpu.make_async_copy(rel_hbm.at[relidx_v.at[0]],
                                  nbufA.at[pl.ds(0, 128)], semA).wait()
            pltpu.sync_copy(nbufA.at[pl.ds(0, 128)],
                            out_rel.at[pl.ds(w * _RPW + t * 128, 128)])

        @pl.when(lax.rem(t, 2) == 1)
        def _():
            pltpu.make_async_copy(rel_hbm.at[relidx_v.at[0]],
                                  nbufB.at[pl.ds(0, 128)], semB).wait()
            pltpu.sync_copy(nbufB.at[pl.ds(0, 128)],
                            out_rel.at[pl.ds(w * _RPW + t * 128, 128)])
        return carry

    lax.fori_loop(0, _RCHUNKS, rel_chunk, 0)


def _sc_gather(idx5, relidx, node_emb, rel_emb):
    mesh = plsc.VectorSubcoreMesh(core_axis_name="c", subcore_axis_name="s")
    return pl.kernel(
        _sc_body,
        out_type=(
            jax.ShapeDtypeStruct((_NE, 2 * _D), jnp.float32),
            jax.ShapeDtypeStruct((_NS, _D), jnp.float32),
        ),
        mesh=mesh,
        scratch_types=[
            pltpu.VMEM((3 * _BATCH, 128), jnp.int32),
            pltpu.VMEM((3 * _BATCH, 128), jnp.int32),
            pltpu.VMEM((_NROWS + _HG, _D), jnp.float32),
            pltpu.VMEM((_NROWS + _HG, _D), jnp.float32),
            pltpu.VMEM((_HG, 2 * _D), jnp.float32),
            pltpu.VMEM((_HG, 2 * _D), jnp.float32),
            pltpu.VMEM((_RCHUNKS, 128), jnp.int32),
            pltpu.SemaphoreType.DMA,
            pltpu.SemaphoreType.DMA,
            pltpu.SemaphoreType.DMA,
            pltpu.SemaphoreType.DMA,
            pltpu.SemaphoreType.DMA,
        ],
    )(idx5, relidx, node_emb, rel_emb)


_BP = 2048
_NBLK = _P // _BP


def _tc_body(enc_sp, enc_sn, enc_dp, enc_dn,
             rel_p, rel_n, wf, bvec, out):
    i = pl.program_id(0)

    def enc(x):
        h = lax.dot_general(x[...], wf[...], (((1,), (0,)), ((), ())),
                            preferred_element_type=jnp.float32)
        return jnp.maximum(h + bvec[...], 0.0)

    hsp = enc(enc_sp)
    hsn = enc(enc_sn)
    hdp = enc(enc_dp)
    hdn = enc(enc_dn)
    dp = hsp + rel_p[...] - hdp
    dn = hsn + rel_n[...] - hdn
    sp = -jnp.sqrt(jnp.sum(dp * dp, axis=1) + 1e-12)
    sn = -jnp.sqrt(jnp.sum(dn * dn, axis=1) + 1e-12)
    part = jnp.sum(jnp.maximum(0.0, sn - sp + _MARGIN)) * (1.0 / _P)

    @pl.when(i == 0)
    def _():
        out[0, 0] = 0.0

    out[0, 0] += part


def _tc_dense(out_enc, out_rel, wf, bvec):
    enc_spec = lambda off: pl.BlockSpec((_BP, 2 * _D),
                                        lambda i, o=off: (i + o, 0))
    rel_spec = lambda off: pl.BlockSpec((_BP, _D),
                                        lambda i, o=off: (i + o, 0))
    loss = pl.pallas_call(
        _tc_body,
        grid=(_NBLK,),
        in_specs=[
            enc_spec(0), enc_spec(_NBLK), enc_spec(2 * _NBLK),
            enc_spec(3 * _NBLK),
            rel_spec(0), rel_spec(_NBLK),
            pl.BlockSpec((2 * _D, _D), lambda i: (0, 0)),
            pl.BlockSpec((1, _D), lambda i: (0, 0)),
        ],
        out_specs=pl.BlockSpec((1, 1), lambda i: (0, 0),
                               memory_space=pltpu.SMEM),
        out_shape=jax.ShapeDtypeStruct((1, 1), jnp.float32),
        compiler_params=pltpu.CompilerParams(
            dimension_semantics=("arbitrary",)),
    )(out_enc, out_enc, out_enc, out_enc,
      out_rel, out_rel, wf, bvec)
    return loss[0, 0]


def kernel(train_pos, train_neg, ngh_idx_src, ngh_idx_dst,
           node_emb, rel_emb, W, b):
    alls = jnp.concatenate([train_pos, train_neg], axis=0).astype(jnp.int32)
    src = alls[:, 0] % _NODE_NUM
    dst = alls[:, 1] % _NODE_NUM
    rel = alls[:, 2] % _REL_NUM

    selfidx = jnp.concatenate([src, dst]).reshape(_NHG, _HG)
    nghidx = jnp.concatenate(
        [ngh_idx_src, ngh_idx_dst], axis=0).astype(jnp.int32).reshape(
            _NHG, _NROWS)
    pad = jnp.zeros((_NHG, 384 - _NROWS - _HG), dtype=jnp.int32)
    idx5 = jnp.concatenate([nghidx, selfidx, pad], axis=1).reshape(
        3 * _NHG, 128)
    relidx = rel.reshape(_NS // 128, 128)

    out_enc, out_rel = _sc_gather(idx5, relidx, node_emb, rel_emb)

    wf = jnp.concatenate([W[:_D], W[_D:] * (1.0 / _K)], axis=0)
    bvec = b.reshape(1, _D)
    return _tc_dense(out_enc, out_rel, wf, bvec)
